# baseline (speedup 1.0000x reference)
import functools

import jax
import jax.numpy as jnp
from jax import lax
from jax.experimental import pallas as pl

N_HEADS, HEAD_DIM, TOPK = 8, 64, 512
EPS = 1e-6
RB = 256
CHUNK = 512


def _rotate_half(x):
    h = x.shape[-1] // 2
    x1, x2 = x[..., :h], x[..., h:]
    return jnp.concatenate([-x2, x1], axis=-1)


def _proj_kernel(x_ref, wq_ref, wk_ref, ww_ref, q_ref, k_ref, w_ref):
    x = x_ref[0]
    s = x.shape[0]
    def _kchunk_dot(a, b, kc=256):
        acc = None
        for c in range(0, a.shape[1], kc):
            p = jnp.dot(a[:, c:c + kc], b[c:c + kc, :],
                        preferred_element_type=jnp.float32)
            acc = p if acc is None else acc + p
        return acc

    q = _kchunk_dot(x, wq_ref[...])
    k = _kchunk_dot(x, wk_ref[...])
    w = _kchunk_dot(x, ww_ref[...])
    w = w * (N_HEADS ** (-0.5) * HEAD_DIM ** (-0.5))
    q_ref[0] = q.reshape(s, N_HEADS, HEAD_DIM)
    k_ref[0] = k
    w_ref[0] = w


def _lex_first(va, ia, vb, ib):
    return (va > vb) | ((va == vb) & (ia < ib))


def _compare_exchange(v, i, d, dir_desc):
    pos = lax.broadcasted_iota(jnp.int32, v.shape, v.ndim - 1)
    is_low = (pos & d) == 0
    pv = jnp.where(is_low, jnp.roll(v, -d, axis=-1), jnp.roll(v, d, axis=-1))
    pi = jnp.where(is_low, jnp.roll(i, -d, axis=-1), jnp.roll(i, d, axis=-1))
    self_first = _lex_first(v, i, pv, pi)
    keep = self_first == (is_low == dir_desc)
    return jnp.where(keep, v, pv), jnp.where(keep, i, pi)


def _bitonic_sort_chunks(v, i, base_desc):
    n = v.shape[-1]
    pos = lax.broadcasted_iota(jnp.int32, v.shape, v.ndim - 1)
    m = 2
    while m <= n:
        dir_desc = base_desc ^ ((pos & m) != 0)
        d = m // 2
        while d >= 1:
            v, i = _compare_exchange(v, i, d, dir_desc)
            d //= 2
        m *= 2
    return v, i


def _bitonic_merge(v, i, dir_desc):
    d = v.shape[-1] // 2
    while d >= 1:
        v, i = _compare_exchange(v, i, d, dir_desc)
        d //= 2
    return v, i


def _topk_block(s):
    rb, t = s.shape
    c = t // CHUNK
    v = s.reshape(rb, c, CHUNK)
    i = (lax.broadcasted_iota(jnp.int32, (rb, c, CHUNK), 1) * CHUNK
         + lax.broadcasted_iota(jnp.int32, (rb, c, CHUNK), 2))
    chunk_par = lax.broadcasted_iota(jnp.int32, (rb, c, CHUNK), 1)
    v, i = _bitonic_sort_chunks(v, i, (chunk_par & 1) == 0)
    while c > 1:
        c //= 2
        va = v.reshape(rb, c, 2, CHUNK)[:, :, 0, :]
        vb = v.reshape(rb, c, 2, CHUNK)[:, :, 1, :]
        ia = i.reshape(rb, c, 2, CHUNK)[:, :, 0, :]
        ib = i.reshape(rb, c, 2, CHUNK)[:, :, 1, :]
        first = _lex_first(va, ia, vb, ib)
        v = jnp.where(first, va, vb)
        i = jnp.where(first, ia, ib)
        chunk_par = lax.broadcasted_iota(jnp.int32, (rb, c, CHUNK), 1)
        v, i = _bitonic_merge(v, i, (chunk_par & 1) == 0)
    return i.reshape(rb, CHUNK)


def _score_topk_kernel(q_ref, k_ref, w_ref, score_ref, idx_ref):
    blk = pl.program_id(1)
    q = q_ref[0]
    k = k_ref[0]
    w = w_ref[0]
    rb = q.shape[0]
    t = k.shape[0]
    qs = q.reshape(rb * N_HEADS, HEAD_DIM)
    sc = lax.dot_general(qs, k, (((1,), (1,)), ((), ())),
                         preferred_element_type=jnp.float32)
    sc = sc.reshape(rb, N_HEADS, t)
    sc = jax.nn.relu(sc)
    sc = (sc * w[:, :, None]).sum(axis=1)
    row = blk * RB + lax.broadcasted_iota(jnp.int32, (rb, t), 0)
    col = lax.broadcasted_iota(jnp.int32, (rb, t), 1)
    sc = jnp.where(col <= row, sc, -jnp.inf)
    score_ref[0] = sc
    idx_ref[0] = _topk_block(sc)


def kernel(x, cos, sin, Wq, Wk, Ww, q_norm_w, k_norm_w, start_pos, end_pos):
    bsz, seqlen, d_model = x.shape

    PB = 512
    q, k, w = pl.pallas_call(
        _proj_kernel,
        grid=(bsz, seqlen // PB),
        in_specs=[
            pl.BlockSpec((1, PB, d_model), lambda b, i: (b, i, 0)),
            pl.BlockSpec((d_model, N_HEADS * HEAD_DIM), lambda b, i: (0, 0)),
            pl.BlockSpec((d_model, HEAD_DIM), lambda b, i: (0, 0)),
            pl.BlockSpec((d_model, N_HEADS), lambda b, i: (0, 0)),
        ],
        out_specs=[
            pl.BlockSpec((1, PB, N_HEADS, HEAD_DIM), lambda b, i: (b, i, 0, 0)),
            pl.BlockSpec((1, PB, HEAD_DIM), lambda b, i: (b, i, 0)),
            pl.BlockSpec((1, PB, N_HEADS), lambda b, i: (b, i, 0)),
        ],
        out_shape=[
            jax.ShapeDtypeStruct((bsz, seqlen, N_HEADS, HEAD_DIM), jnp.float32),
            jax.ShapeDtypeStruct((bsz, seqlen, HEAD_DIM), jnp.float32),
            jax.ShapeDtypeStruct((bsz, seqlen, N_HEADS), jnp.float32),
        ],
    )(x, Wq, Wk, Ww)

    var = jnp.mean(q * q, axis=-1, keepdims=True)
    q = (q * lax.rsqrt(var + EPS)) * q_norm_w
    var = jnp.mean(k * k, axis=-1, keepdims=True)
    k = (k * lax.rsqrt(var + EPS)) * k_norm_w
    q = q * cos[:, :, None, :] + _rotate_half(q) * sin[:, :, None, :]
    k = k * cos + _rotate_half(k) * sin

    score, topk_indices = pl.pallas_call(
        _score_topk_kernel,
        grid=(bsz, seqlen // RB),
        in_specs=[
            pl.BlockSpec((1, RB, N_HEADS, HEAD_DIM), lambda b, i: (b, i, 0, 0)),
            pl.BlockSpec((1, seqlen, HEAD_DIM), lambda b, i: (b, 0, 0)),
            pl.BlockSpec((1, RB, N_HEADS), lambda b, i: (b, i, 0)),
        ],
        out_specs=[
            pl.BlockSpec((1, RB, seqlen), lambda b, i: (b, i, 0)),
            pl.BlockSpec((1, RB, TOPK), lambda b, i: (b, i, 0)),
        ],
        out_shape=[
            jax.ShapeDtypeStruct((bsz, seqlen, seqlen), jnp.float32),
            jax.ShapeDtypeStruct((bsz, seqlen, TOPK), jnp.int32),
        ],
    )(q, k, w)

    return topk_indices, score


# speedup vs baseline: 1.0486x; 1.0486x over previous
import jax
import jax.numpy as jnp
from jax import lax
from jax.experimental import pallas as pl

N_HEADS, HEAD_DIM, TOPK = 8, 64, 512
EPS = 1e-6
RB = 256
CHUNK = 512


def _rotate_half(x):
    h = x.shape[-1] // 2
    x1, x2 = x[..., :h], x[..., h:]
    return jnp.concatenate([-x2, x1], axis=-1)


def _proj_kernel(x_ref, wq_ref, wk_ref, ww_ref, q_ref, k_ref, w_ref):
    x = x_ref[0]
    s = x.shape[0]
    q = jnp.dot(x, wq_ref[...], preferred_element_type=jnp.float32)
    k = jnp.dot(x, wk_ref[...], preferred_element_type=jnp.float32)
    w = jnp.dot(x, ww_ref[...], preferred_element_type=jnp.float32)
    w = w * (N_HEADS ** (-0.5) * HEAD_DIM ** (-0.5))
    q_ref[0] = q.reshape(s, N_HEADS, HEAD_DIM)
    k_ref[0] = k
    w_ref[0] = w


def _lex_first(va, ia, vb, ib):
    return (va > vb) | ((va == vb) & (ia < ib))


def _base_desc(chunk_iota, odd3):
    if odd3:
        return chunk_iota < 1
    return (chunk_iota & 1) == 0


def _ce(v, i, d, m, odd3=False):
    rb, c, n = v.shape
    pos = lax.broadcasted_iota(jnp.int32, (rb, c, n), 2)
    is_low = (pos & d) == 0
    pv = jnp.where(is_low, jnp.roll(v, -d, axis=2), jnp.roll(v, d, axis=2))
    pi = jnp.where(is_low, jnp.roll(i, -d, axis=2), jnp.roll(i, d, axis=2))
    self_first = _lex_first(v, i, pv, pi)
    chunk = lax.broadcasted_iota(jnp.int32, (rb, c, n), 1)
    seg_odd = ((pos // m) & 1) == 1
    dir_desc = _base_desc(chunk, odd3) ^ seg_odd
    keep = self_first == (is_low == dir_desc)
    return jnp.where(keep, v, pv), jnp.where(keep, i, pi)


def _bitonic_sort_chunks(v, i, odd3):
    n = v.shape[2]
    m = 2
    while m <= n:
        d = m // 2
        while d >= 1:
            v, i = _ce(v, i, d, m, odd3)
            d //= 2
        m *= 2
    return v, i


def _bitonic_merge(v, i):
    n = v.shape[2]
    d = n // 2
    while d >= 1:
        v, i = _ce(v, i, d, 2 * n)
        d //= 2
    return v, i


def _winners(va, ia, vb, ib):
    a_first = _lex_first(va, ia, vb, ib)
    return jnp.where(a_first, va, vb), jnp.where(a_first, ia, ib)


def _topk(s):
    rb, t = s.shape
    c = t // CHUNK
    v = s.reshape(rb, c, CHUNK)
    i = (lax.broadcasted_iota(jnp.int32, (rb, c, CHUNK), 1) * CHUNK
         + lax.broadcasted_iota(jnp.int32, (rb, c, CHUNK), 2))
    v, i = _bitonic_sort_chunks(v, i, odd3=(c == 3))
    while c > 1:
        if c & 1:
            wv, wi = _winners(v[:, 0:1], i[:, 0:1], v[:, 1:2], i[:, 1:2])
            wv, wi = _bitonic_merge(wv, wi)
            v = jnp.concatenate([wv, v[:, 2:]], axis=1)
            i = jnp.concatenate([wi, i[:, 2:]], axis=1)
            c -= 1
        else:
            c //= 2
            vr = v.reshape(rb, c, 2, CHUNK)
            ir = i.reshape(rb, c, 2, CHUNK)
            v, i = _winners(vr[:, :, 0], ir[:, :, 0], vr[:, :, 1], ir[:, :, 1])
            v, i = _bitonic_merge(v, i)
    return i.reshape(rb, CHUNK)


def _make_score_topk(nchunks, seqlen):
    t_c = nchunks * CHUNK

    def _kernel(q_ref, k_ref, w_ref, score_ref, idx_ref):
        blk = pl.program_id(1)
        q = q_ref[0]
        k = k_ref[0]
        w = w_ref[0]
        rb = q.shape[0]
        qs = q.reshape(rb * N_HEADS, HEAD_DIM)
        sc = lax.dot_general(qs, k, (((1,), (1,)), ((), ())),
                             preferred_element_type=jnp.float32)
        sc = sc.reshape(rb, N_HEADS, t_c)
        sc = jax.nn.relu(sc)
        sc = (sc * w[:, :, None]).sum(axis=1)
        row0 = (2 * (nchunks - 1) + blk) * RB
        row = row0 + lax.broadcasted_iota(jnp.int32, (rb, t_c), 0)
        col = lax.broadcasted_iota(jnp.int32, (rb, t_c), 1)
        sc = jnp.where(col <= row, sc, -jnp.inf)
        if t_c < seqlen:
            score_ref[0] = jnp.concatenate(
                [sc, jnp.full((rb, seqlen - t_c), -jnp.inf, jnp.float32)],
                axis=1)
        else:
            score_ref[0] = sc
        idx_ref[0] = _topk(sc)

    return _kernel


def kernel(x, cos, sin, Wq, Wk, Ww, q_norm_w, k_norm_w, start_pos, end_pos):
    bsz, seqlen, d_model = x.shape

    PB = 512
    q, k, w = pl.pallas_call(
        _proj_kernel,
        grid=(bsz, seqlen // PB),
        in_specs=[
            pl.BlockSpec((1, PB, d_model), lambda b, i: (b, i, 0)),
            pl.BlockSpec((d_model, N_HEADS * HEAD_DIM), lambda b, i: (0, 0)),
            pl.BlockSpec((d_model, HEAD_DIM), lambda b, i: (0, 0)),
            pl.BlockSpec((d_model, N_HEADS), lambda b, i: (0, 0)),
        ],
        out_specs=[
            pl.BlockSpec((1, PB, N_HEADS, HEAD_DIM), lambda b, i: (b, i, 0, 0)),
            pl.BlockSpec((1, PB, HEAD_DIM), lambda b, i: (b, i, 0)),
            pl.BlockSpec((1, PB, N_HEADS), lambda b, i: (b, i, 0)),
        ],
        out_shape=[
            jax.ShapeDtypeStruct((bsz, seqlen, N_HEADS, HEAD_DIM), jnp.float32),
            jax.ShapeDtypeStruct((bsz, seqlen, HEAD_DIM), jnp.float32),
            jax.ShapeDtypeStruct((bsz, seqlen, N_HEADS), jnp.float32),
        ],
    )(x, Wq, Wk, Ww)

    var = jnp.mean(q * q, axis=-1, keepdims=True)
    q = (q * lax.rsqrt(var + EPS)) * q_norm_w
    var = jnp.mean(k * k, axis=-1, keepdims=True)
    k = (k * lax.rsqrt(var + EPS)) * k_norm_w
    q = q * cos[:, :, None, :] + _rotate_half(q) * sin[:, :, None, :]
    k = k * cos + _rotate_half(k) * sin

    score_parts = []
    idx_parts = []
    for c in (1, 2, 3, 4):
        body = _make_score_topk(c, seqlen)
        sc_c, ix_c = pl.pallas_call(
            body,
            grid=(bsz, 2),
            in_specs=[
                pl.BlockSpec((1, RB, N_HEADS, HEAD_DIM),
                             lambda b, j, c=c: (b, 2 * (c - 1) + j, 0, 0)),
                pl.BlockSpec((1, c * CHUNK, HEAD_DIM),
                             lambda b, j: (b, 0, 0)),
                pl.BlockSpec((1, RB, N_HEADS),
                             lambda b, j, c=c: (b, 2 * (c - 1) + j, 0)),
            ],
            out_specs=[
                pl.BlockSpec((1, RB, seqlen), lambda b, j: (b, j, 0)),
                pl.BlockSpec((1, RB, TOPK), lambda b, j: (b, j, 0)),
            ],
            out_shape=[
                jax.ShapeDtypeStruct((bsz, 2 * RB, seqlen), jnp.float32),
                jax.ShapeDtypeStruct((bsz, 2 * RB, TOPK), jnp.int32),
            ],
        )(q, k, w)
        score_parts.append(sc_c)
        idx_parts.append(ix_c)

    score = jnp.concatenate(score_parts, axis=1)
    topk_indices = jnp.concatenate(idx_parts, axis=1)
    return topk_indices, score
